# baseline (speedup 1.0000x reference)
import functools

import jax
import jax.numpy as jnp
from jax import lax
from jax.experimental import pallas as pl
from jax.experimental.pallas import tpu as pltpu
from jax.experimental.pallas import tpu_sc as plsc

BATCH = 4096
SEQ_LEN = 200
EMBED_DIM = 64
NUM_ROWS = BATCH * SEQ_LEN

NC = 2
NS = 16
NW = NC * NS
LANES = 16

ROWS_PER_WORKER = NUM_ROWS // NW
SEQS_PER_CHUNK = 2
CHUNK = SEQS_PER_CHUNK * SEQ_LEN
NCHUNK = ROWS_PER_WORKER // CHUNK


def _make_kernel():
  mesh = plsc.VectorSubcoreMesh(
      core_axis_name="c", subcore_axis_name="s",
      num_cores=NC, num_subcores=NS)

  @functools.partial(
      pl.kernel,
      out_type=jax.ShapeDtypeStruct((NUM_ROWS, EMBED_DIM), jnp.float32),
      mesh=mesh,
      scratch_types=[
          pltpu.VMEM((CHUNK,), jnp.int32),
          pltpu.VMEM((CHUNK, EMBED_DIM), jnp.float32),
          pltpu.VMEM((SEQ_LEN, EMBED_DIM), jnp.float32),
          pltpu.SemaphoreType.DMA,
      ],
      compiler_params=pltpu.CompilerParams(use_tc_tiling_on_sc=False),
  )
  def emb_kernel(idx_hbm, table_hbm, pos_hbm, out_hbm,
                 idx_v, rows_v, pos_v, sem):
    wid = lax.axis_index("s") * NC + lax.axis_index("c")
    wbase = wid * ROWS_PER_WORKER

    pltpu.sync_copy(pos_hbm.at[pl.ds(0, SEQ_LEN)], pos_v)

    @pl.loop(0, NCHUNK)
    def chunk_loop(g):
      base = wbase + g * CHUNK
      pltpu.sync_copy(idx_hbm.at[pl.ds(base, CHUNK)], idx_v)
      pltpu.async_copy(table_hbm.at[idx_v], rows_v, sem).wait()

      @pl.loop(0, SEQ_LEN)
      def pos_loop(p):
        for q in range(EMBED_DIM // LANES):
          col = pl.ds(q * LANES, LANES)
          pv = pos_v[p, col]
          for s in range(SEQS_PER_CHUNK):
            rows_v[s * SEQ_LEN + p, col] += pv

      pltpu.sync_copy(rows_v, out_hbm.at[pl.ds(base, CHUNK)])

  return emb_kernel


_emb_kernel = _make_kernel()


def kernel(x, word_table, pos_table):
  xf = x.reshape(NUM_ROWS).astype(jnp.int32)
  out = _emb_kernel(xf, word_table, pos_table)
  return out.reshape(BATCH, SEQ_LEN, EMBED_DIM)


# speedup vs baseline: 1.1268x; 1.1268x over previous
import functools

import jax
import jax.numpy as jnp
from jax import lax
from jax.experimental import pallas as pl
from jax.experimental.pallas import tpu as pltpu
from jax.experimental.pallas import tpu_sc as plsc

BATCH = 4096
SEQ_LEN = 200
EMBED_DIM = 64
NUM_ROWS = BATCH * SEQ_LEN

NC = 2
NS = 16
NW = NC * NS
LANES = 16

ROWS_PER_WORKER = NUM_ROWS // NW
SEQS_PER_CHUNK = 2
CHUNK = SEQS_PER_CHUNK * SEQ_LEN
NCHUNK = ROWS_PER_WORKER // CHUNK
NBUF = 4


def _make_kernel():
  mesh = plsc.VectorSubcoreMesh(
      core_axis_name="c", subcore_axis_name="s",
      num_cores=NC, num_subcores=NS)

  @functools.partial(
      pl.kernel,
      out_type=jax.ShapeDtypeStruct((NUM_ROWS, EMBED_DIM), jnp.float32),
      mesh=mesh,
      scratch_types=[
          pltpu.VMEM((NBUF, CHUNK), jnp.int32),
          pltpu.VMEM((NBUF, CHUNK, EMBED_DIM), jnp.float32),
          pltpu.VMEM((SEQ_LEN, EMBED_DIM), jnp.float32),
          [pltpu.SemaphoreType.DMA] * NBUF,
          [pltpu.SemaphoreType.DMA] * NBUF,
          [pltpu.SemaphoreType.DMA] * NBUF,
      ],
      compiler_params=pltpu.CompilerParams(use_tc_tiling_on_sc=False),
  )
  def emb_kernel(idx_hbm, table_hbm, pos_hbm, out_hbm,
                 idx_v, rows_v, pos_v, isem, gsem, wsem):
    wid = lax.axis_index("s") * NC + lax.axis_index("c")
    wbase = wid * ROWS_PER_WORKER

    def idx_start(c, slot):
      pltpu.async_copy(idx_hbm.at[pl.ds(wbase + c * CHUNK, CHUNK)],
                       idx_v.at[slot], isem[slot])

    def idx_wait(slot):
      pltpu.make_async_copy(idx_hbm.at[pl.ds(0, CHUNK)],
                            idx_v.at[slot], isem[slot]).wait()

    def gather_start(slot):
      pltpu.async_copy(table_hbm.at[idx_v.at[slot]], rows_v.at[slot],
                       gsem[slot])

    def gather_wait(slot):
      pltpu.make_async_copy(table_hbm.at[idx_v.at[slot]], rows_v.at[slot],
                            gsem[slot]).wait()

    def wb_start(c, slot):
      pltpu.async_copy(rows_v.at[slot],
                       out_hbm.at[pl.ds(wbase + c * CHUNK, CHUNK)],
                       wsem[slot])

    def wb_wait(slot):
      pltpu.make_async_copy(rows_v.at[slot],
                            out_hbm.at[pl.ds(0, CHUNK)], wsem[slot]).wait()

    def pos_add(slot):
      @pl.loop(0, SEQ_LEN)
      def pos_loop(p):
        for q in range(EMBED_DIM // LANES):
          col = pl.ds(q * LANES, LANES)
          pv = pos_v[p, col]
          for s in range(SEQS_PER_CHUNK):
            rows_v[slot, s * SEQ_LEN + p, col] += pv

    def stage(i, b, do_wb_wait, do_idx, do_gather):
      if do_wb_wait:
        wb_wait((b + 2) % NBUF)
      if do_idx:
        idx_start(i + 3, (b + 3) % NBUF)
      if do_gather:
        idx_wait((b + 2) % NBUF)
        gather_start((b + 2) % NBUF)
      gather_wait(b)
      pos_add(b)
      wb_start(i, b)

    pltpu.sync_copy(pos_hbm.at[pl.ds(0, SEQ_LEN)], pos_v)

    for c in range(3):
      idx_start(c, c)
    for c in range(2):
      idx_wait(c)
      gather_start(c)

    stage(0, 0, False, True, True)
    stage(1, 1, False, True, True)
    stage(2, 2, True, True, True)
    stage(3, 3, True, True, True)

    @pl.loop(4, NCHUNK - 4, step=NBUF)
    def main_loop(i):
      for b in range(NBUF):
        stage(i + b, b, True, True, True)

    stage(NCHUNK - 4, 0, True, True, True)
    stage(NCHUNK - 3, 1, True, False, True)
    stage(NCHUNK - 2, 2, True, False, False)
    stage(NCHUNK - 1, 3, True, False, False)
    wb_wait(2)
    wb_wait(3)

  return emb_kernel


_emb_kernel = _make_kernel()


def kernel(x, word_table, pos_table):
  xf = x.reshape(NUM_ROWS).astype(jnp.int32)
  out = _emb_kernel(xf, word_table, pos_table)
  return out.reshape(BATCH, SEQ_LEN, EMBED_DIM)


# speedup vs baseline: 1.1301x; 1.0029x over previous
import functools

import jax
import jax.numpy as jnp
from jax import lax
from jax.experimental import pallas as pl
from jax.experimental.pallas import tpu as pltpu
from jax.experimental.pallas import tpu_sc as plsc

BATCH = 4096
SEQ_LEN = 200
EMBED_DIM = 64
NUM_ROWS = BATCH * SEQ_LEN

NC = 2
NS = 16
NW = NC * NS

ROWS_PER_WORKER = NUM_ROWS // NW
SEQS_PER_CHUNK = 2
CHUNK = SEQS_PER_CHUNK * SEQ_LEN
NCHUNK = ROWS_PER_WORKER // CHUNK
NBUF = 4


def _make_kernel():
  mesh = plsc.VectorSubcoreMesh(
      core_axis_name="c", subcore_axis_name="s",
      num_cores=NC, num_subcores=NS)

  @functools.partial(
      pl.kernel,
      out_type=jax.ShapeDtypeStruct((NUM_ROWS, EMBED_DIM), jnp.float32),
      mesh=mesh,
      scratch_types=[
          pltpu.VMEM((NBUF, CHUNK), jnp.int32),
          pltpu.VMEM((NBUF, CHUNK, EMBED_DIM), jnp.float32),
          pltpu.VMEM_SHARED((CHUNK, EMBED_DIM), jnp.float32),
          [pltpu.SemaphoreType.DMA] * NBUF,
          [pltpu.SemaphoreType.DMA] * NBUF,
          [pltpu.SemaphoreType.DMA] * NBUF,
          [pltpu.SemaphoreType.DMA] * NBUF,
      ],
      compiler_params=pltpu.CompilerParams(use_tc_tiling_on_sc=False),
  )
  def emb_kernel(idx_hbm, table_hbm, pos_hbm, out_hbm,
                 idx_v, rows_v, pos_sh, isem, fsem, gsem, wsem):
    cid = lax.axis_index("c")
    sid = lax.axis_index("s")
    wid = sid * NC + cid
    wbase = wid * ROWS_PER_WORKER

    def idx_start(c, slot):
      pltpu.async_copy(idx_hbm.at[pl.ds(wbase + c * CHUNK, CHUNK)],
                       idx_v.at[slot], isem[slot])

    def idx_wait(slot):
      pltpu.make_async_copy(idx_hbm.at[pl.ds(0, CHUNK)],
                            idx_v.at[slot], isem[slot]).wait()

    def fill_start(slot):
      pltpu.async_copy(pos_sh, rows_v.at[slot], fsem[slot])

    def fill_wait(slot):
      pltpu.make_async_copy(pos_sh, rows_v.at[slot], fsem[slot]).wait()

    def gather_start(slot):
      pltpu.async_copy(table_hbm.at[idx_v.at[slot]], rows_v.at[slot],
                       gsem[slot], add=True)

    def gather_wait(slot):
      pltpu.make_async_copy(table_hbm.at[idx_v.at[slot]], rows_v.at[slot],
                            gsem[slot]).wait()

    def wb_start(c, slot):
      pltpu.async_copy(rows_v.at[slot],
                       out_hbm.at[pl.ds(wbase + c * CHUNK, CHUNK)],
                       wsem[slot])

    def wb_wait(slot):
      pltpu.make_async_copy(rows_v.at[slot],
                            out_hbm.at[pl.ds(0, CHUNK)], wsem[slot]).wait()

    def stage(i, b, do_wb_wait, do_idx, do_gather):
      if do_wb_wait:
        wb_wait((b + 2) % NBUF)
      if do_gather:
        fill_start((b + 2) % NBUF)
      if do_idx:
        idx_start(i + 3, (b + 3) % NBUF)
      if do_gather:
        idx_wait((b + 2) % NBUF)
        fill_wait((b + 2) % NBUF)
        gather_start((b + 2) % NBUF)
      gather_wait(b)
      wb_start(i, b)

    @pl.when(sid == 0)
    def _():
      for s in range(SEQS_PER_CHUNK):
        pltpu.sync_copy(pos_hbm.at[pl.ds(0, SEQ_LEN)],
                        pos_sh.at[pl.ds(s * SEQ_LEN, SEQ_LEN)])
    plsc.subcore_barrier()

    for c in range(3):
      idx_start(c, c)
    for c in range(2):
      fill_start(c)
    for c in range(2):
      idx_wait(c)
      fill_wait(c)
      gather_start(c)

    stage(0, 0, False, True, True)
    stage(1, 1, False, True, True)
    stage(2, 2, True, True, True)
    stage(3, 3, True, True, True)

    @pl.loop(4, NCHUNK - 4, step=NBUF)
    def main_loop(i):
      for b in range(NBUF):
        stage(i + b, b, True, True, True)

    stage(NCHUNK - 4, 0, True, True, True)
    stage(NCHUNK - 3, 1, True, False, True)
    stage(NCHUNK - 2, 2, True, False, False)
    stage(NCHUNK - 1, 3, True, False, False)
    wb_wait(2)
    wb_wait(3)

  return emb_kernel


_emb_kernel = _make_kernel()


def kernel(x, word_table, pos_table):
  xf = x.reshape(NUM_ROWS).astype(jnp.int32)
  out = _emb_kernel(xf, word_table, pos_table)
  return out.reshape(BATCH, SEQ_LEN, EMBED_DIM)
